# baseline (speedup 1.0000x reference)
import functools

import numpy as np
import jax
import jax.numpy as jnp
from jax import lax
from jax.experimental import pallas as pl
from jax.experimental.pallas import tpu as pltpu
from jax.experimental.pallas import tpu_sc as plsc

_NUM_BUCKETS = 100000
_HIDDEN = 128
_MAX_T = 0.7
_INV_MAX_T = float(np.float32(1.0) / np.float32(_MAX_T))
_B = 16384

_NC = 2
_NS = 16
_L = 16
_NW = _NC * _NS
_BPW = _B // _NW
_CHUNK = 64
_NCHUNK = _BPW // _CHUNK

_mesh = plsc.VectorSubcoreMesh(core_axis_name="c", subcore_axis_name="s")


@functools.partial(
    pl.kernel,
    out_type=jax.ShapeDtypeStruct((_B, _HIDDEN), jnp.float32),
    mesh=_mesh,
    scratch_types=[
        pltpu.VMEM((_BPW,), jnp.float32),
        pltpu.VMEM((2, _HIDDEN), jnp.float32),
        pltpu.VMEM((_BPW, _HIDDEN), jnp.float32),
        pltpu.SemaphoreType.DMA,
        pltpu.SemaphoreType.DMA,
    ],
)
def _embed(x_hbm, table_hbm, out_hbm, x_v, r01_v, rows_v, in_sem, out_sem):
    wid = lax.axis_index("s") * _NC + lax.axis_index("c")
    base = wid * _BPW

    in_copies = [
        pltpu.async_copy(x_hbm.at[pl.ds(base, _BPW)], x_v, in_sem),
        pltpu.async_copy(table_hbm.at[pl.ds(0, 1)], r01_v.at[pl.ds(0, 1)],
                         in_sem),
        pltpu.async_copy(table_hbm.at[pl.ds(_NUM_BUCKETS - 1, 1)],
                         r01_v.at[pl.ds(1, 1)], in_sem),
    ]
    for cp in in_copies:
        cp.wait()

    r0 = [r01_v[0, pl.ds(k * _L, _L)] for k in range(_HIDDEN // _L)]
    r1 = [r01_v[1, pl.ds(k * _L, _L)] for k in range(_HIDDEN // _L)]

    def body(g, carry):
        e0 = g * _L
        xv = x_v[pl.ds(e0, _L)]
        t = jnp.minimum(jnp.maximum(xv * _INV_MAX_T, 0.0), 1.0) * _NUM_BUCKETS
        c16 = jnp.minimum(t.astype(jnp.int32), 1).astype(jnp.float32)
        for e in range(_L):
            c = lax.broadcast_in_dim(c16[e], (_L,), ())
            a = 1.0 - c
            for k in range(_HIDDEN // _L):
                rows_v[e0 + e, pl.ds(k * _L, _L)] = r0[k] * a + r1[k] * c
        return carry

    copies = []
    for j in range(_NCHUNK):
        lax.fori_loop(j * (_CHUNK // _L), (j + 1) * (_CHUNK // _L), body, 0)
        copies.append(pltpu.async_copy(
            rows_v.at[pl.ds(j * _CHUNK, _CHUNK)],
            out_hbm.at[pl.ds(base + j * _CHUNK, _CHUNK)],
            out_sem,
        ))
    for cp in copies:
        cp.wait()


def kernel(x, table):
    return _embed(x, table)


# speedup vs baseline: 1.1328x; 1.1328x over previous
import functools

import numpy as np
import jax
import jax.numpy as jnp
from jax import lax
from jax.experimental import pallas as pl
from jax.experimental.pallas import tpu as pltpu
from jax.experimental.pallas import tpu_sc as plsc

_NUM_BUCKETS = 100000
_HIDDEN = 128
_MAX_T = 0.7
_INV_MAX_T = float(np.float32(1.0) / np.float32(_MAX_T))
_B = 16384

_NC = 2
_NS = 16
_L = 16
_NW = _NC * _NS
_BPW = _B // _NW
_CHUNK = 128
_NCHUNK = _BPW // _CHUNK

_mesh = plsc.VectorSubcoreMesh(core_axis_name="c", subcore_axis_name="s")


@functools.partial(
    pl.kernel,
    out_type=jax.ShapeDtypeStruct((_B, _HIDDEN), jnp.float32),
    mesh=_mesh,
    scratch_types=[
        pltpu.VMEM((_BPW,), jnp.float32),
        pltpu.VMEM((2, _HIDDEN), jnp.float32),
        pltpu.VMEM((_BPW, _HIDDEN), jnp.float32),
        pltpu.SemaphoreType.DMA,
        pltpu.SemaphoreType.DMA,
    ],
)
def _embed(x_hbm, table_hbm, out_hbm, x_v, r01_v, rows_v, in_sem, out_sem):
    wid = lax.axis_index("s") * _NC + lax.axis_index("c")
    base = wid * _BPW

    in_copies = [
        pltpu.async_copy(x_hbm.at[pl.ds(base, _BPW)], x_v, in_sem),
        pltpu.async_copy(table_hbm.at[pl.ds(0, 1)], r01_v.at[pl.ds(0, 1)],
                         in_sem),
        pltpu.async_copy(table_hbm.at[pl.ds(_NUM_BUCKETS - 1, 1)],
                         r01_v.at[pl.ds(1, 1)], in_sem),
    ]
    for cp in in_copies:
        cp.wait()

    r0 = [r01_v[0, pl.ds(k * _L, _L)] for k in range(_HIDDEN // _L)]
    r1 = [r01_v[1, pl.ds(k * _L, _L)] for k in range(_HIDDEN // _L)]

    def body(g, carry):
        e0 = g * _L
        xv = x_v[pl.ds(e0, _L)]
        t = jnp.minimum(jnp.maximum(xv * _INV_MAX_T, 0.0), 1.0) * _NUM_BUCKETS
        c16 = jnp.minimum(t.astype(jnp.int32), 1).astype(jnp.float32)
        for e in range(_L):
            c = lax.broadcast_in_dim(c16[e], (_L,), ())
            a = 1.0 - c
            for k in range(_HIDDEN // _L):
                rows_v[e0 + e, pl.ds(k * _L, _L)] = r0[k] * a + r1[k] * c
        return carry

    copies = []
    for j in range(_NCHUNK):
        lax.fori_loop(j * (_CHUNK // _L), (j + 1) * (_CHUNK // _L), body, 0)
        copies.append(pltpu.async_copy(
            rows_v.at[pl.ds(j * _CHUNK, _CHUNK)],
            out_hbm.at[pl.ds(base + j * _CHUNK, _CHUNK)],
            out_sem,
        ))
    for cp in copies:
        cp.wait()


def kernel(x, table):
    return _embed(x, table)


# speedup vs baseline: 1.1444x; 1.0102x over previous
import functools

import numpy as np
import jax
import jax.numpy as jnp
from jax import lax
from jax.experimental import pallas as pl
from jax.experimental.pallas import tpu as pltpu
from jax.experimental.pallas import tpu_sc as plsc

_NUM_BUCKETS = 100000
_HIDDEN = 128
_MAX_T = 0.7
_INV_MAX_T = float(np.float32(1.0) / np.float32(_MAX_T))
_B = 16384

_NC = 2
_NS = 16
_L = 16
_NW = _NC * _NS
_BPW = _B // _NW
_CHUNK = 128
_NCHUNK = _BPW // _CHUNK

_mesh = plsc.VectorSubcoreMesh(core_axis_name="c", subcore_axis_name="s")


@functools.partial(
    pl.kernel,
    out_type=jax.ShapeDtypeStruct((_B, _HIDDEN), jnp.float32),
    mesh=_mesh,
    scratch_types=[
        pltpu.VMEM((_BPW,), jnp.float32),
        pltpu.VMEM((2, _HIDDEN), jnp.float32),
        pltpu.VMEM((_BPW, _HIDDEN), jnp.float32),
        pltpu.SemaphoreType.DMA,
        pltpu.SemaphoreType.DMA,
    ],
)
def _embed(x_hbm, table_hbm, out_hbm, x_v, r01_v, rows_v, in_sem, out_sem):
    wid = lax.axis_index("s") * _NC + lax.axis_index("c")
    base = wid * _BPW

    in_copies = [
        pltpu.async_copy(x_hbm.at[pl.ds(base, _BPW)], x_v, in_sem),
        pltpu.async_copy(table_hbm.at[pl.ds(0, 1)], r01_v.at[pl.ds(0, 1)],
                         in_sem),
        pltpu.async_copy(table_hbm.at[pl.ds(_NUM_BUCKETS - 1, 1)],
                         r01_v.at[pl.ds(1, 1)], in_sem),
    ]
    for cp in in_copies:
        cp.wait()

    nk = _HIDDEN // _L
    rr = ([r01_v[0, pl.ds(k * _L, _L)] for k in range(nk)]
          + [r01_v[1, pl.ds(k * _L, _L)] for k in range(nk)])

    def body(g, carry):
        e0 = g * _L
        xv = x_v[pl.ds(e0, _L)]
        t = jnp.minimum(jnp.maximum(xv * _INV_MAX_T, 0.0), 1.0) * _NUM_BUCKETS
        c16 = jnp.minimum(t.astype(jnp.int32), 1).astype(jnp.float32)
        for e in range(_L):
            c = lax.broadcast_in_dim(c16[e], (_L,), ())
            a = 1.0 - c
            for k in range(nk):
                rows_v[e0 + e, pl.ds(k * _L, _L)] = (
                    carry[k] * a + carry[nk + k] * c)
        return carry

    copies = []
    for j in range(_NCHUNK):
        lax.fori_loop(j * (_CHUNK // _L), (j + 1) * (_CHUNK // _L), body, rr,
                      unroll=2)
        copies.append(pltpu.async_copy(
            rows_v.at[pl.ds(j * _CHUNK, _CHUNK)],
            out_hbm.at[pl.ds(base + j * _CHUNK, _CHUNK)],
            out_sem,
        ))
    for cp in copies:
        cp.wait()


def kernel(x, table):
    return _embed(x, table)
